# Initial kernel scaffold; baseline (speedup 1.0000x reference)
import jax
import jax.numpy as jnp
from jax.experimental import pallas as pl


def kernel(x, table):
    raise NotImplementedError("write your pallas kernel here")


# speedup vs baseline: 3.2421x; 3.2421x over previous
import functools

import jax
import jax.numpy as jnp
from jax import lax
from jax.experimental import pallas as pl
from jax.experimental.pallas import tpu as pltpu
from jax.experimental.pallas import tpu_sc as plsc

N_EMB = 100000
DIM = 128
BATCH, SEQ = 4096, 50
B_TOTAL = BATCH * SEQ

NUM_CORES = 2
NUM_SUBCORES = 16
NW = NUM_CORES * NUM_SUBCORES
ROWS_PER_W = B_TOTAL // NW
CHUNK = 128
NCH = ROWS_PER_W // CHUNK
NBUF = 2
NT = NCH // NBUF

_mesh = plsc.VectorSubcoreMesh(core_axis_name="c", subcore_axis_name="s")


@functools.partial(
    pl.kernel,
    mesh=_mesh,
    out_type=jax.ShapeDtypeStruct((B_TOTAL, DIM), jnp.float32),
    scratch_types=[
        pltpu.VMEM((NCH, CHUNK), jnp.int32),
        pltpu.VMEM((NBUF, CHUNK, DIM), jnp.float32),
        pltpu.SemaphoreType.DMA,
        pltpu.SemaphoreType.DMA,
        pltpu.SemaphoreType.DMA,
        pltpu.SemaphoreType.DMA,
    ],
)
def _sc_gather(idx_hbm, table_hbm, out_hbm, idx_v, bufs, g0, g1, s0, s1):
    gsems = (g0, g1)
    ssems = (s0, s1)
    wid = lax.axis_index("s") * NUM_CORES + lax.axis_index("c")
    base = wid * ROWS_PER_W

    pltpu.sync_copy(idx_hbm.at[wid], idx_v)

    def gather_start(j, b):
        pltpu.async_copy(table_hbm.at[idx_v.at[j]], bufs.at[b], gsems[b])

    def gather_wait(b):
        pltpu.make_async_copy(
            table_hbm.at[idx_v.at[0]], bufs.at[b], gsems[b]).wait()

    def store_start(j, b):
        pltpu.async_copy(
            bufs.at[b], out_hbm.at[pl.ds(base + j * CHUNK, CHUNK)], ssems[b])

    def store_wait(b):
        pltpu.make_async_copy(
            bufs.at[b], out_hbm.at[pl.ds(base, CHUNK)], ssems[b]).wait()

    for b in range(NBUF):
        gather_start(b, b)

    def body(t, _):
        for b in range(NBUF):
            j = t * NBUF + b
            gather_wait(b)
            store_start(j, b)
        for b in range(NBUF):
            j = t * NBUF + b
            store_wait(b)
            gather_start(j + NBUF, b)
        return ()

    lax.fori_loop(0, NT - 1, body, (), unroll=False)

    for b in range(NBUF):
        j = (NT - 1) * NBUF + b
        gather_wait(b)
        store_start(j, b)
    for b in range(NBUF):
        store_wait(b)


def kernel(x, table):
    idx = x.reshape(NW, NCH, CHUNK).astype(jnp.int32)
    out = _sc_gather(idx, table)
    return out.reshape(BATCH, SEQ, DIM)


# speedup vs baseline: 3.2986x; 1.0174x over previous
import functools

import jax
import jax.numpy as jnp
from jax import lax
from jax.experimental import pallas as pl
from jax.experimental.pallas import tpu as pltpu
from jax.experimental.pallas import tpu_sc as plsc

N_EMB = 100000
DIM = 128
BATCH, SEQ = 4096, 50
B_TOTAL = BATCH * SEQ

NUM_CORES = 2
NUM_SUBCORES = 16
NW = NUM_CORES * NUM_SUBCORES
ROWS_PER_W = B_TOTAL // NW
CHUNK = 128
NCH = ROWS_PER_W // CHUNK
NBUF = 5
NT = NCH // NBUF

_mesh = plsc.VectorSubcoreMesh(core_axis_name="c", subcore_axis_name="s")


@functools.partial(
    pl.kernel,
    mesh=_mesh,
    out_type=jax.ShapeDtypeStruct((B_TOTAL, DIM), jnp.float32),
    scratch_types=[
        pltpu.VMEM((NCH, CHUNK), jnp.int32),
        pltpu.VMEM((NBUF, CHUNK, DIM), jnp.float32),
    ] + [pltpu.SemaphoreType.DMA] * (2 * NBUF),
)
def _sc_gather(idx_hbm, table_hbm, out_hbm, idx_v, bufs, *sems):
    gsems = sems[:NBUF]
    ssems = sems[NBUF:]
    wid = lax.axis_index("s") * NUM_CORES + lax.axis_index("c")
    base = wid * ROWS_PER_W

    pltpu.sync_copy(idx_hbm.at[wid], idx_v)

    def gather_start(j, b):
        pltpu.async_copy(table_hbm.at[idx_v.at[j]], bufs.at[b], gsems[b])

    def gather_wait(b):
        pltpu.make_async_copy(
            table_hbm.at[idx_v.at[0]], bufs.at[b], gsems[b]).wait()

    def store_start(j, b):
        pltpu.async_copy(
            bufs.at[b], out_hbm.at[pl.ds(base + j * CHUNK, CHUNK)], ssems[b])

    def store_wait(b):
        pltpu.make_async_copy(
            bufs.at[b], out_hbm.at[pl.ds(base, CHUNK)], ssems[b]).wait()

    for b in range(NBUF):
        gather_start(b, b)

    def body(t, _):
        for b in range(NBUF):
            j = t * NBUF + b
            gather_wait(b)
            store_start(j, b)
        for b in range(NBUF):
            j = t * NBUF + b
            store_wait(b)
            gather_start(j + NBUF, b)
        return ()

    lax.fori_loop(0, NT - 1, body, (), unroll=False)

    for b in range(NBUF):
        j = (NT - 1) * NBUF + b
        gather_wait(b)
        store_start(j, b)
    for b in range(NBUF):
        store_wait(b)


def kernel(x, table):
    idx = x.reshape(NW, NCH, CHUNK).astype(jnp.int32)
    out = _sc_gather(idx, table)
    return out.reshape(BATCH, SEQ, DIM)


# speedup vs baseline: 5.8907x; 1.7858x over previous
import functools

import jax
import jax.numpy as jnp
from jax import lax
from jax.experimental import pallas as pl
from jax.experimental.pallas import tpu as pltpu
from jax.experimental.pallas import tpu_sc as plsc

N_EMB = 100000
DIM = 128
BATCH, SEQ = 4096, 50

NUM_CORES = 2
NUM_SUBCORES = 16
NW = NUM_CORES * NUM_SUBCORES
ROWS_PER_W = BATCH // NW
GRP = 8
NG = ROWS_PER_W // GRP

_mesh = plsc.VectorSubcoreMesh(core_axis_name="c", subcore_axis_name="s")


@functools.partial(
    pl.kernel,
    mesh=_mesh,
    out_type=jax.ShapeDtypeStruct((BATCH, SEQ, DIM), jnp.float32),
    scratch_types=[
        pltpu.VMEM((2, GRP, SEQ), jnp.int32),
        pltpu.VMEM((2, GRP, SEQ, DIM), jnp.float32),
        pltpu.SemaphoreType.DMA,
        pltpu.SemaphoreType.DMA,
        pltpu.SemaphoreType.DMA,
    ],
)
def _sc_gather(x_hbm, table_hbm, out_hbm, idx_v, bufs, isem, gsem, ssem):
    wid = lax.axis_index("s") * NUM_CORES + lax.axis_index("c")
    base = wid * ROWS_PER_W

    def idx_start(g, slot):
        pltpu.async_copy(
            x_hbm.at[pl.ds(base + g * GRP, GRP)], idx_v.at[slot], isem)

    def idx_wait(slot):
        pltpu.make_async_copy(
            x_hbm.at[pl.ds(base, GRP)], idx_v.at[slot], isem).wait()

    def gathers_start(slot):
        for r in range(GRP):
            pltpu.async_copy(
                table_hbm.at[idx_v.at[slot, r]], bufs.at[slot, r], gsem)

    def gathers_wait(slot):
        for r in range(GRP):
            pltpu.make_async_copy(
                table_hbm.at[idx_v.at[slot, 0]], bufs.at[slot, r], gsem).wait()

    def store_start(g, slot):
        pltpu.async_copy(
            bufs.at[slot], out_hbm.at[pl.ds(base + g * GRP, GRP)], ssem)

    def store_wait(slot):
        pltpu.make_async_copy(
            bufs.at[slot], out_hbm.at[pl.ds(base, GRP)], ssem).wait()

    idx_start(0, 0)
    idx_start(1, 1)
    for g in (0, 1):
        slot = g
        idx_wait(slot)
        gathers_start(slot)
        gathers_wait(slot)
        store_start(g, slot)
        idx_start(g + 2, slot)

    def body(g, _):
        slot = g & 1
        idx_wait(slot)
        store_wait(slot)
        gathers_start(slot)
        gathers_wait(slot)
        store_start(g, slot)
        idx_start(g + 2, slot)
        return ()

    lax.fori_loop(2, NG - 2, body, (), unroll=False)

    for g in (NG - 2, NG - 1):
        slot = g & 1
        idx_wait(slot)
        store_wait(slot)
        gathers_start(slot)
        gathers_wait(slot)
        store_start(g, slot)
    for slot in (0, 1):
        store_wait(slot)


def kernel(x, table):
    return _sc_gather(x.astype(jnp.int32), table)


# speedup vs baseline: 10.3391x; 1.7552x over previous
import functools

import jax
import jax.numpy as jnp
from jax import lax
from jax.experimental import pallas as pl
from jax.experimental.pallas import tpu as pltpu
from jax.experimental.pallas import tpu_sc as plsc

N_EMB = 100000
DIM = 128
BATCH, SEQ = 4096, 50

NUM_CORES = 2
NUM_SUBCORES = 16
NW = NUM_CORES * NUM_SUBCORES
BSLAB = BATCH // NW
HALF = BSLAB // 2
NBUF = 5
NT = SEQ // NBUF

_mesh = plsc.VectorSubcoreMesh(core_axis_name="c", subcore_axis_name="s")


@functools.partial(
    pl.kernel,
    mesh=_mesh,
    out_type=jax.ShapeDtypeStruct((SEQ, BATCH, DIM), jnp.float32),
    scratch_types=[
        pltpu.VMEM((SEQ, BSLAB), jnp.int32),
        pltpu.VMEM((NBUF, BSLAB, DIM), jnp.float32),
    ] + [pltpu.SemaphoreType.DMA] * (2 * NBUF),
)
def _sc_gather(xt_hbm, table_hbm, out_hbm, idx_v, bufs, *sems):
    gsems = sems[:NBUF]
    ssems = sems[NBUF:]
    wid = lax.axis_index("s") * NUM_CORES + lax.axis_index("c")
    col0 = wid * BSLAB

    pltpu.sync_copy(xt_hbm.at[:, pl.ds(col0, BSLAB)], idx_v)

    def gathers_start(s, b):
        for h in range(2):
            pltpu.async_copy(
                table_hbm.at[idx_v.at[s, pl.ds(h * HALF, HALF)]],
                bufs.at[b, pl.ds(h * HALF, HALF)], gsems[b])

    def gathers_wait(b):
        for h in range(2):
            pltpu.make_async_copy(
                table_hbm.at[idx_v.at[0, pl.ds(0, HALF)]],
                bufs.at[b, pl.ds(h * HALF, HALF)], gsems[b]).wait()

    def store_start(s, b):
        pltpu.async_copy(
            bufs.at[b], out_hbm.at[s, pl.ds(col0, BSLAB)], ssems[b])

    def store_wait(b):
        pltpu.make_async_copy(
            bufs.at[b], out_hbm.at[0, pl.ds(col0, BSLAB)], ssems[b]).wait()

    for b in range(NBUF):
        gathers_start(b, b)

    def body(t, _):
        for b in range(NBUF):
            s = t * NBUF + b
            gathers_wait(b)
            store_start(s, b)
        for b in range(NBUF):
            s = t * NBUF + b
            store_wait(b)
            gathers_start(s + NBUF, b)
        return ()

    lax.fori_loop(0, NT - 1, body, (), unroll=False)

    for b in range(NBUF):
        s = (NT - 1) * NBUF + b
        gathers_wait(b)
        store_start(s, b)
    for b in range(NBUF):
        store_wait(b)


def kernel(x, table):
    xt = x.T.astype(jnp.int32)
    out_t = _sc_gather(xt, table)
    return jnp.transpose(out_t, (1, 0, 2))
